# baseline (speedup 1.0000x reference)
import functools

import jax
import jax.numpy as jnp
from jax import lax
from jax.experimental import pallas as pl
from jax.experimental.pallas import tpu as pltpu
from jax.experimental.pallas import tpu_sc as plsc

NC = 2
NS = 16
NW = NC * NS
L = 16
R = 128
NBUF = 3


def _sc_body(nodes, wexp, out, xbuf, wbuf, ostage, xsem, wsem, *, B, N, F):
    fpw = F // (NW // B)
    nt = fpw // L
    nchunks = N // R

    cid = lax.axis_index("c")
    sid = lax.axis_index("s")
    w = sid * NC + cid
    b = w // (NW // B)
    fs = (w % (NW // B)) * fpw

    def x_copy(ci, slot):
        return pltpu.make_async_copy(
            nodes.at[b, pl.ds(ci * R, R), pl.ds(fs, fpw)],
            xbuf.at[slot],
            xsem.at[slot],
        )

    def w_copy(ci, slot):
        return pltpu.make_async_copy(
            wexp.at[b, pl.ds(ci * R, R), :],
            wbuf.at[slot],
            wsem.at[slot],
        )

    for k in range(NBUF):
        x_copy(k, k).start()
        w_copy(k, k).start()

    def chunk_body(ci, accs):
        slot = lax.rem(ci, NBUF)
        x_copy(ci, slot).wait()
        w_copy(ci, slot).wait()

        def node_fma(n, accs):
            wv = wbuf[slot, n]
            return tuple(
                accs[t] + xbuf[slot, n, pl.ds(t * L, L)] * wv
                for t in range(nt)
            )

        accs = lax.fori_loop(0, R, node_fma, accs)

        nxt = ci + NBUF

        @pl.when(nxt < nchunks)
        def _():
            x_copy(nxt, slot).start()
            w_copy(nxt, slot).start()

        return accs

    accs0 = tuple(jnp.zeros((L,), jnp.float32) for _ in range(nt))
    accs = lax.fori_loop(0, nchunks, chunk_body, accs0)

    for t in range(nt):
        ostage[pl.ds(t * L, L)] = accs[t]
    pltpu.sync_copy(ostage, out.at[b, pl.ds(fs, fpw)])


def kernel(nodes, weights):
    B, N, F = nodes.shape
    fpw = F // (NW // B)
    wexp = jnp.broadcast_to(weights * (1.0 / N), (B, N, L))

    mesh = plsc.VectorSubcoreMesh(
        core_axis_name="c", subcore_axis_name="s",
        num_cores=NC, num_subcores=NS,
    )
    k = pl.kernel(
        functools.partial(_sc_body, B=B, N=N, F=F),
        out_type=jax.ShapeDtypeStruct((B, F), jnp.float32),
        mesh=mesh,
        scratch_types=[
            pltpu.VMEM((NBUF, R, fpw), jnp.float32),
            pltpu.VMEM((NBUF, R, L), jnp.float32),
            pltpu.VMEM((fpw,), jnp.float32),
            pltpu.SemaphoreType.DMA((NBUF,)),
            pltpu.SemaphoreType.DMA((NBUF,)),
        ],
        compiler_params=pltpu.CompilerParams(use_tc_tiling_on_sc=False),
    )
    return k(nodes, wexp)


# speedup vs baseline: 1.0112x; 1.0112x over previous
import functools

import jax
import jax.numpy as jnp
from jax import lax
from jax.experimental import pallas as pl
from jax.experimental.pallas import tpu as pltpu
from jax.experimental.pallas import tpu_sc as plsc

NC = 2
NS = 16
NW = NC * NS
L = 16
R = 64
NBUF = 4


def _sc_body(nodes, wexp, out, xbuf, wbuf, ostage, xsem, wsem, *, B, N, F):
    fpw = F // (NW // B)
    nt = fpw // L
    nchunks = N // R
    ngroups = nchunks // NBUF

    cid = lax.axis_index("c")
    sid = lax.axis_index("s")
    w = sid * NC + cid
    b = w // (NW // B)
    fs = (w % (NW // B)) * fpw

    def x_copy(ci, slot):
        return pltpu.make_async_copy(
            nodes.at[b, pl.ds(ci * R, R), pl.ds(fs, fpw)],
            xbuf.at[slot],
            xsem.at[slot],
        )

    def w_copy(ci, slot):
        return pltpu.make_async_copy(
            wexp.at[b, pl.ds(ci * R, R), :],
            wbuf.at[slot],
            wsem.at[slot],
        )

    for k in range(NBUF):
        x_copy(k, k).start()
        w_copy(k, k).start()

    def group_body(g, accs):
        for k in range(NBUF):
            ci = g * NBUF + k
            x_copy(ci, k).wait()
            w_copy(ci, k).wait()

            def node_fma(n, accs, k=k):
                wv = wbuf.at[k][n]
                return tuple(
                    accs[t] + xbuf.at[k][n, pl.ds(t * L, L)] * wv
                    for t in range(nt)
                )

            accs = lax.fori_loop(0, R, node_fma, accs)

            nxt = ci + NBUF

            @pl.when(nxt < nchunks)
            def _(ci=ci, k=k):
                x_copy(ci + NBUF, k).start()
                w_copy(ci + NBUF, k).start()
        return accs

    accs0 = tuple(jnp.zeros((L,), jnp.float32) for _ in range(nt))
    accs = lax.fori_loop(0, ngroups, group_body, accs0)

    for t in range(nt):
        ostage[pl.ds(t * L, L)] = accs[t]
    pltpu.sync_copy(ostage, out.at[b, pl.ds(fs, fpw)])


def kernel(nodes, weights):
    B, N, F = nodes.shape
    fpw = F // (NW // B)
    wexp = jnp.broadcast_to(weights * (1.0 / N), (B, N, L))

    mesh = plsc.VectorSubcoreMesh(
        core_axis_name="c", subcore_axis_name="s",
        num_cores=NC, num_subcores=NS,
    )
    k = pl.kernel(
        functools.partial(_sc_body, B=B, N=N, F=F),
        out_type=jax.ShapeDtypeStruct((B, F), jnp.float32),
        mesh=mesh,
        scratch_types=[
            pltpu.VMEM((NBUF, R, fpw), jnp.float32),
            pltpu.VMEM((NBUF, R, L), jnp.float32),
            pltpu.VMEM((fpw,), jnp.float32),
            pltpu.SemaphoreType.DMA((NBUF,)),
            pltpu.SemaphoreType.DMA((NBUF,)),
        ],
        compiler_params=pltpu.CompilerParams(use_tc_tiling_on_sc=False),
    )
    return k(nodes, wexp)


# speedup vs baseline: 1.7550x; 1.7356x over previous
import functools

import jax
import jax.numpy as jnp
from jax import lax
from jax.experimental import pallas as pl
from jax.experimental.pallas import tpu as pltpu
from jax.experimental.pallas import tpu_sc as plsc

NC = 2
NS = 16
NW = NC * NS
L = 16
R = 64
NBUF = 4


def _sc_body(nodes, wexp, out, xbuf, wbuf, ostage, xsem, wsem, *, B, N, F):
    fpw = F // (NW // B)
    nt = fpw // L
    nchunks = N // R
    ngroups = nchunks // NBUF

    cid = lax.axis_index("c")
    sid = lax.axis_index("s")
    w = sid * NC + cid
    b = w // (NW // B)
    fs = (w % (NW // B)) * fpw

    def x_copy(ci, slot):
        return pltpu.make_async_copy(
            nodes.at[b, pl.ds(ci * R, R), pl.ds(fs, fpw)],
            xbuf.at[slot],
            xsem.at[slot],
        )

    def w_copy(ci, slot):
        return pltpu.make_async_copy(
            wexp.at[b, pl.ds(ci * R, R), :],
            wbuf.at[slot],
            wsem.at[slot],
        )

    for k in range(NBUF):
        x_copy(k, k).start()
        w_copy(k, k).start()

    def group_body(g, accs):
        for k in range(NBUF):
            ci = g * NBUF + k
            x_copy(ci, k).wait()
            w_copy(ci, k).wait()

            def node_fma(n, accs, k=k):
                wv = wbuf.at[k][n]
                return tuple(
                    accs[t] + xbuf.at[k][n, pl.ds(t * L, L)] * wv
                    for t in range(nt)
                )

            accs = lax.fori_loop(0, R, node_fma, accs)

            nxt = ci + NBUF

            @pl.when(nxt < nchunks)
            def _(ci=ci, k=k):
                x_copy(ci + NBUF, k).start()
                w_copy(ci + NBUF, k).start()
        return accs

    accs0 = tuple(jnp.zeros((L,), jnp.float32) for _ in range(nt))
    accs = lax.fori_loop(0, ngroups, group_body, accs0)

    for t in range(nt):
        ostage[pl.ds(t * L, L)] = accs[t]
    pltpu.sync_copy(ostage, out.at[b, pl.ds(fs, fpw)])


def kernel(nodes, weights):
    B, N, F = nodes.shape
    fpw = F // (NW // B)
    wexp = jnp.broadcast_to(weights * (1.0 / N), (B, N, L))

    mesh = plsc.VectorSubcoreMesh(
        core_axis_name="c", subcore_axis_name="s",
        num_cores=NC, num_subcores=NS,
    )
    k = pl.kernel(
        functools.partial(_sc_body, B=B, N=N, F=F),
        out_type=jax.ShapeDtypeStruct((B, F), jnp.float32),
        mesh=mesh,
        scratch_types=[
            pltpu.VMEM((NBUF, R, fpw), jnp.float32),
            pltpu.VMEM((NBUF, R, L), jnp.float32),
            pltpu.VMEM((fpw,), jnp.float32),
            pltpu.SemaphoreType.DMA((NBUF,)),
            pltpu.SemaphoreType.DMA((NBUF,)),
        ],
        compiler_params=pltpu.CompilerParams(use_tc_tiling_on_sc=True),
    )
    return k(nodes, wexp)
